# baseline (speedup 1.0000x reference)
import functools

import jax
import jax.numpy as jnp
from jax import lax
from jax.experimental import pallas as pl
from jax.experimental.pallas import tpu as pltpu
from jax.experimental.pallas import tpu_sc as plsc

DIM = 128
HALF = 64
CHUNK = 128
NBUF = 6
LANES = 16


def _build_sc_gather(n_tokens: int):
    info = plsc.get_sparse_core_info()
    nc, ns = info.num_cores, info.num_subcores
    nw = nc * ns
    b_per_w = n_tokens // nw
    assert n_tokens % nw == 0 and b_per_w % CHUNK == 0
    n_chunks = b_per_w // CHUNK

    mesh = plsc.VectorSubcoreMesh(core_axis_name="c", subcore_axis_name="s")
    out = jax.ShapeDtypeStruct((2 * n_tokens, HALF), jnp.float32)

    @functools.partial(
        pl.kernel,
        mesh=mesh,
        out_type=(out, out),
        compiler_params=pltpu.CompilerParams(use_tc_tiling_on_sc=False),
        scratch_types=[
            pltpu.VMEM((n_chunks, CHUNK), jnp.int32),
            pltpu.VMEM((n_chunks, CHUNK), jnp.int32),
            pltpu.VMEM((n_chunks, CHUNK), jnp.int32),
            pltpu.VMEM((NBUF, CHUNK, HALF), jnp.float32),
            pltpu.VMEM((NBUF, CHUNK, HALF), jnp.float32),
            pltpu.SemaphoreType.DMA,
            pltpu.SemaphoreType.DMA,
            pltpu.SemaphoreType.DMA,
            pltpu.SemaphoreType.DMA,
        ],
    )
    def gather_kernel(pos_hbm, cos_hbm, sin_hbm, cos_out, sin_out,
                      gidx_v, weven_v, wodd_v, cos_rows, sin_rows,
                      sem_gc, sem_gs, sem_wc, sem_ws):
        wid = lax.axis_index("s") * nc + lax.axis_index("c")
        pltpu.sync_copy(pos_hbm.at[pl.ds(wid * n_chunks, n_chunks)], gidx_v)

        def double_chunk(c):
            def body(g, _):
                sl = pl.ds(g * LANES, LANES)
                gidx_v[c, sl] = gidx_v[c, sl] * 2
                return 0
            lax.fori_loop(0, CHUNK // LANES, body, 0, unroll=4)

        def issue_gather(c):
            b = c % NBUF
            return (
                pltpu.async_copy(cos_hbm.at[gidx_v.at[c]], cos_rows.at[b], sem_gc),
                pltpu.async_copy(sin_hbm.at[gidx_v.at[c]], sin_rows.at[b], sem_gs),
            )

        def issue_write(c):
            b = c % NBUF
            return (
                pltpu.async_copy(cos_rows.at[b], cos_out.at[weven_v.at[c]], sem_wc),
                pltpu.async_copy(cos_rows.at[b], cos_out.at[wodd_v.at[c]], sem_wc),
                pltpu.async_copy(sin_rows.at[b], sin_out.at[weven_v.at[c]], sem_ws),
                pltpu.async_copy(sin_rows.at[b], sin_out.at[wodd_v.at[c]], sem_ws),
            )

        double_chunk(0)
        gathers = {0: issue_gather(0)}
        for c in range(1, n_chunks):
            double_chunk(c)
        for c in range(1, min(NBUF - 1, n_chunks)):
            gathers[c] = issue_gather(c)
        vbase = lax.iota(jnp.int32, LANES) * 2 + (2 * b_per_w) * wid
        for c in range(n_chunks):
            def wbody(g, _, c=c):
                v = vbase + (2 * c * CHUNK + 2 * LANES * g)
                sl = pl.ds(g * LANES, LANES)
                weven_v[c, sl] = v
                wodd_v[c, sl] = v + 1
                return 0
            lax.fori_loop(0, CHUNK // LANES, wbody, 0, unroll=4)

        writes = {}
        for c in range(n_chunks):
            g = c + NBUF - 1
            if g < n_chunks:
                if g - NBUF >= 0:
                    for op in writes.pop(g - NBUF):
                        op.wait()
                gathers[g] = issue_gather(g)
            for op in gathers.pop(c):
                op.wait()
            writes[c] = issue_write(c)
        for c in sorted(writes):
            for op in writes.pop(c):
                op.wait()

    return gather_kernel


def kernel(x, position_ids, cos_cached, sin_cached):
    b, s = position_ids.shape
    n_tokens = b * s
    pos2d = position_ids.astype(jnp.int32).reshape(n_tokens // CHUNK, CHUNK)
    cos_half = cos_cached.reshape(-1, HALF)
    sin_half = sin_cached.reshape(-1, HALF)
    gather = _build_sc_gather(n_tokens)
    cos_flat, sin_flat = gather(pos2d, cos_half, sin_half)
    cos = cos_flat.reshape(b, s, DIM).astype(x.dtype)
    sin = sin_flat.reshape(b, s, DIM).astype(x.dtype)
    return (cos, sin)


# speedup vs baseline: 1.0216x; 1.0216x over previous
import functools

import jax
import jax.numpy as jnp
from jax import lax
from jax.experimental import pallas as pl
from jax.experimental.pallas import tpu as pltpu
from jax.experimental.pallas import tpu_sc as plsc

DIM = 128
HALF = 64
CHUNK = 256
NBUF = 3
LANES = 16


def _build_sc_gather(n_tokens: int):
    info = plsc.get_sparse_core_info()
    nc, ns = info.num_cores, info.num_subcores
    nw = nc * ns
    b_per_w = n_tokens // nw
    assert n_tokens % nw == 0 and b_per_w % CHUNK == 0
    n_chunks = b_per_w // CHUNK

    mesh = plsc.VectorSubcoreMesh(core_axis_name="c", subcore_axis_name="s")
    out = jax.ShapeDtypeStruct((2 * n_tokens, HALF), jnp.float32)

    @functools.partial(
        pl.kernel,
        mesh=mesh,
        out_type=(out, out),
        compiler_params=pltpu.CompilerParams(use_tc_tiling_on_sc=False),
        scratch_types=[
            pltpu.VMEM((n_chunks, CHUNK), jnp.int32),
            pltpu.VMEM((n_chunks, CHUNK), jnp.int32),
            pltpu.VMEM((n_chunks, CHUNK), jnp.int32),
            pltpu.VMEM((NBUF, CHUNK, HALF), jnp.float32),
            pltpu.VMEM((NBUF, CHUNK, HALF), jnp.float32),
            pltpu.SemaphoreType.DMA,
            pltpu.SemaphoreType.DMA,
            pltpu.SemaphoreType.DMA,
            pltpu.SemaphoreType.DMA,
        ],
    )
    def gather_kernel(pos_hbm, cos_hbm, sin_hbm, cos_out, sin_out,
                      gidx_v, weven_v, wodd_v, cos_rows, sin_rows,
                      sem_gc, sem_gs, sem_wc, sem_ws):
        wid = lax.axis_index("s") * nc + lax.axis_index("c")
        pltpu.sync_copy(pos_hbm.at[pl.ds(wid * n_chunks, 1)], gidx_v.at[pl.ds(0, 1)])

        def double_chunk(c):
            def body(g, _):
                sl = pl.ds(g * LANES, LANES)
                gidx_v[c, sl] = gidx_v[c, sl] * 2
                return 0
            lax.fori_loop(0, CHUNK // LANES, body, 0, unroll=4)

        def issue_gather(c):
            b = c % NBUF
            return (
                pltpu.async_copy(cos_hbm.at[gidx_v.at[c]], cos_rows.at[b], sem_gc),
                pltpu.async_copy(sin_hbm.at[gidx_v.at[c]], sin_rows.at[b], sem_gs),
            )

        def issue_write(c):
            b = c % NBUF
            return (
                pltpu.async_copy(cos_rows.at[b], cos_out.at[weven_v.at[c]], sem_wc),
                pltpu.async_copy(cos_rows.at[b], cos_out.at[wodd_v.at[c]], sem_wc),
                pltpu.async_copy(sin_rows.at[b], sin_out.at[weven_v.at[c]], sem_ws),
                pltpu.async_copy(sin_rows.at[b], sin_out.at[wodd_v.at[c]], sem_ws),
            )

        double_chunk(0)
        gathers = {0: issue_gather(0)}
        pltpu.sync_copy(pos_hbm.at[pl.ds(wid * n_chunks + 1, n_chunks - 1)],
                        gidx_v.at[pl.ds(1, n_chunks - 1)])
        for c in range(1, n_chunks):
            double_chunk(c)
            if c < NBUF - 1:
                gathers[c] = issue_gather(c)
        vbase = lax.iota(jnp.int32, LANES) * 2 + (2 * b_per_w) * wid
        for c in range(n_chunks):
            def wbody(g, _, c=c):
                v = vbase + (2 * c * CHUNK + 2 * LANES * g)
                sl = pl.ds(g * LANES, LANES)
                weven_v[c, sl] = v
                wodd_v[c, sl] = v + 1
                return 0
            lax.fori_loop(0, CHUNK // LANES, wbody, 0, unroll=4)

        writes = {}
        for c in range(n_chunks):
            g = c + NBUF - 1
            if g < n_chunks:
                if g - NBUF >= 0:
                    for op in writes.pop(g - NBUF):
                        op.wait()
                gathers[g] = issue_gather(g)
            for op in gathers.pop(c):
                op.wait()
            writes[c] = issue_write(c)
        for c in sorted(writes):
            for op in writes.pop(c):
                op.wait()

    return gather_kernel


def kernel(x, position_ids, cos_cached, sin_cached):
    b, s = position_ids.shape
    n_tokens = b * s
    pos2d = position_ids.astype(jnp.int32).reshape(n_tokens // CHUNK, CHUNK)
    cos_half = cos_cached.reshape(-1, HALF)
    sin_half = sin_cached.reshape(-1, HALF)
    gather = _build_sc_gather(n_tokens)
    cos_flat, sin_flat = gather(pos2d, cos_half, sin_half)
    cos = cos_flat.reshape(b, s, DIM).astype(x.dtype)
    sin = sin_flat.reshape(b, s, DIM).astype(x.dtype)
    return (cos, sin)


# speedup vs baseline: 1.0248x; 1.0032x over previous
import functools

import jax
import jax.numpy as jnp
from jax import lax
from jax.experimental import pallas as pl
from jax.experimental.pallas import tpu as pltpu
from jax.experimental.pallas import tpu_sc as plsc

DIM = 128
HALF = 64
CHUNK = 256
NBUF = 3
LANES = 16


def _build_sc_gather(n_tokens: int):
    info = plsc.get_sparse_core_info()
    nc, ns = info.num_cores, info.num_subcores
    nw = nc * ns
    b_per_w = n_tokens // nw
    assert n_tokens % nw == 0 and b_per_w % CHUNK == 0
    n_chunks = b_per_w // CHUNK

    mesh = plsc.VectorSubcoreMesh(core_axis_name="c", subcore_axis_name="s")
    out = jax.ShapeDtypeStruct((2 * n_tokens, HALF), jnp.float32)

    @functools.partial(
        pl.kernel,
        mesh=mesh,
        out_type=(out, out),
        compiler_params=pltpu.CompilerParams(use_tc_tiling_on_sc=False,
                                             skip_device_barrier=True),
        scratch_types=[
            pltpu.VMEM((n_chunks, CHUNK), jnp.int32),
            pltpu.VMEM((n_chunks, CHUNK), jnp.int32),
            pltpu.VMEM((n_chunks, CHUNK), jnp.int32),
            pltpu.VMEM((NBUF, CHUNK, HALF), jnp.float32),
            pltpu.VMEM((NBUF, CHUNK, HALF), jnp.float32),
            pltpu.SemaphoreType.DMA,
            pltpu.SemaphoreType.DMA,
            pltpu.SemaphoreType.DMA,
            pltpu.SemaphoreType.DMA,
        ],
    )
    def gather_kernel(pos_hbm, cos_hbm, sin_hbm, cos_out, sin_out,
                      gidx_v, weven_v, wodd_v, cos_rows, sin_rows,
                      sem_gc, sem_gs, sem_wc, sem_ws):
        wid = lax.axis_index("s") * nc + lax.axis_index("c")
        pltpu.sync_copy(pos_hbm.at[pl.ds(wid * n_chunks, 1)], gidx_v.at[pl.ds(0, 1)])

        def double_chunk(c):
            def body(g, _):
                sl = pl.ds(g * LANES, LANES)
                gidx_v[c, sl] = gidx_v[c, sl] * 2
                return 0
            lax.fori_loop(0, CHUNK // LANES, body, 0, unroll=4)

        def issue_gather(c):
            b = c % NBUF
            return (
                pltpu.async_copy(cos_hbm.at[gidx_v.at[c]], cos_rows.at[b], sem_gc),
                pltpu.async_copy(sin_hbm.at[gidx_v.at[c]], sin_rows.at[b], sem_gs),
            )

        def issue_write(c):
            b = c % NBUF
            return (
                pltpu.async_copy(cos_rows.at[b], cos_out.at[weven_v.at[c]], sem_wc),
                pltpu.async_copy(cos_rows.at[b], cos_out.at[wodd_v.at[c]], sem_wc),
                pltpu.async_copy(sin_rows.at[b], sin_out.at[weven_v.at[c]], sem_ws),
                pltpu.async_copy(sin_rows.at[b], sin_out.at[wodd_v.at[c]], sem_ws),
            )

        double_chunk(0)
        gathers = {0: issue_gather(0)}
        pltpu.sync_copy(pos_hbm.at[pl.ds(wid * n_chunks + 1, n_chunks - 1)],
                        gidx_v.at[pl.ds(1, n_chunks - 1)])
        for c in range(1, n_chunks):
            double_chunk(c)
            if c < NBUF - 1:
                gathers[c] = issue_gather(c)
        vbase = lax.iota(jnp.int32, LANES) * 2 + (2 * b_per_w) * wid
        for c in range(n_chunks):
            def wbody(g, _, c=c):
                v = vbase + (2 * c * CHUNK + 2 * LANES * g)
                sl = pl.ds(g * LANES, LANES)
                weven_v[c, sl] = v
                wodd_v[c, sl] = v + 1
                return 0
            lax.fori_loop(0, CHUNK // LANES, wbody, 0, unroll=4)

        writes = {}
        for c in range(n_chunks):
            g = c + NBUF - 1
            if g < n_chunks:
                if g - NBUF >= 0:
                    for op in writes.pop(g - NBUF):
                        op.wait()
                gathers[g] = issue_gather(g)
            for op in gathers.pop(c):
                op.wait()
            writes[c] = issue_write(c)
        for c in sorted(writes):
            for op in writes.pop(c):
                op.wait()

    return gather_kernel


def kernel(x, position_ids, cos_cached, sin_cached):
    b, s = position_ids.shape
    n_tokens = b * s
    pos2d = position_ids.astype(jnp.int32).reshape(n_tokens // CHUNK, CHUNK)
    cos_half = cos_cached.reshape(-1, HALF)
    sin_half = sin_cached.reshape(-1, HALF)
    gather = _build_sc_gather(n_tokens)
    cos_flat, sin_flat = gather(pos2d, cos_half, sin_half)
    cos = cos_flat.reshape(b, s, DIM).astype(x.dtype)
    sin = sin_flat.reshape(b, s, DIM).astype(x.dtype)
    return (cos, sin)
